# baseline (speedup 1.0000x reference)
import functools

import numpy as np
import jax
import jax.numpy as jnp
from jax import lax
from jax.experimental import pallas as pl
from jax.experimental.pallas import tpu as pltpu
from jax.experimental.pallas import tpu_sc as plsc

_MASK_SIZE = 4
_MASK_TOKEN = 14
_B, _N = 16384, 128
_BR = 1024


def _np_uniform_key42(shape):
    n = int(np.prod(shape))
    k0 = np.uint32(0)
    k1 = np.uint32(42)
    ks2 = np.uint32(k0 ^ k1 ^ np.uint32(0x1BD11BDA))
    x0 = np.zeros(n, dtype=np.uint32)
    x1 = np.arange(n, dtype=np.uint32)

    def rotl(x, r):
        return ((x << np.uint32(r)) | (x >> np.uint32(32 - r))).astype(np.uint32)

    def rounds(x0, x1, rots):
        for r in rots:
            x0 = (x0 + x1).astype(np.uint32)
            x1 = rotl(x1, r)
            x1 = x1 ^ x0
        return x0, x1

    ra, rb = (13, 15, 26, 6), (17, 29, 16, 24)
    x0 = (x0 + k0).astype(np.uint32)
    x1 = (x1 + k1).astype(np.uint32)
    x0, x1 = rounds(x0, x1, ra)
    x0 = (x0 + k1).astype(np.uint32); x1 = (x1 + ks2 + np.uint32(1)).astype(np.uint32)
    x0, x1 = rounds(x0, x1, rb)
    x0 = (x0 + ks2).astype(np.uint32); x1 = (x1 + k0 + np.uint32(2)).astype(np.uint32)
    x0, x1 = rounds(x0, x1, ra)
    x0 = (x0 + k0).astype(np.uint32); x1 = (x1 + k1 + np.uint32(3)).astype(np.uint32)
    x0, x1 = rounds(x0, x1, rb)
    x0 = (x0 + k1).astype(np.uint32); x1 = (x1 + ks2 + np.uint32(4)).astype(np.uint32)
    x0, x1 = rounds(x0, x1, ra)
    x0 = (x0 + ks2).astype(np.uint32); x1 = (x1 + k0 + np.uint32(5)).astype(np.uint32)
    bits = x0 ^ x1
    fbits = ((bits >> np.uint32(9)) | np.uint32(0x3F800000)).view(np.float32)
    return (fbits - np.float32(1.0)).reshape(shape)


@functools.lru_cache(maxsize=1)
def _tables():
    u = _np_uniform_key42((_B, _N))
    order = np.argsort(-u, axis=1, kind="stable")
    rank = np.empty((_B, _N), np.int32)
    rank[np.arange(_B)[:, None], order] = np.arange(_N)[None, :]
    big = 10**6
    m = np.full((_MASK_SIZE, _B), big, np.int64)
    thr = np.empty((_B, _N), np.int64)
    for c in range(1, _N + 1):
        x = rank[:, c - 1].astype(np.int64)
        for k in range(_MASK_SIZE):
            lo = np.minimum(m[k], x)
            x = np.maximum(m[k], x)
            m[k] = lo
        thr[:, c - 1] = np.where(m[_MASK_SIZE - 1] >= big, _N - 1, m[_MASK_SIZE - 1])
    return jnp.asarray(rank.astype(np.int8)), jnp.asarray(thr.astype(np.int8))


def _body(ids_ref, attn_ref, rank_ref, thr_ref, oid_ref, omask_ref, xm_ref):
    a = attn_ref[...]
    ones = jnp.ones((_N, _N), dtype=jnp.bfloat16)
    dn = (((1,), (0,)), ((), ()))
    csum = lax.dot_general(a.astype(jnp.bfloat16), ones, dn,
                           preferred_element_type=jnp.float32)
    c = jnp.minimum(csum.astype(jnp.int32) + 1, _N)
    pos = lax.broadcasted_iota(jnp.int32, (_BR, _N), 1)
    tsel = jnp.where(pos == c - 1, thr_ref[...].astype(jnp.bfloat16), jnp.bfloat16(0))
    thr_b = lax.dot_general(tsel, ones, dn,
                            preferred_element_type=jnp.float32).astype(jnp.int32)
    r = rank_ref[...].astype(jnp.int32)
    xm = ((pos < c) & (r <= thr_b)) | ((pos >= c) & (pos < _MASK_SIZE))
    oid_ref[...] = jnp.where(xm, _MASK_TOKEN, ids_ref[...])
    omask_ref[...] = jnp.where(xm, 0, a)
    xm_ref[...] = xm


def _tc_kernel(input_ids, attention_mask):
    rank8, thr8 = _tables()
    spec = pl.BlockSpec((_BR, _N), lambda i: (i, 0))
    out_ids, out_mask, xmask = pl.pallas_call(
        _body,
        grid=(_B // _BR,),
        in_specs=[spec, spec, spec, spec],
        out_specs=[spec, spec, spec],
        out_shape=[
            jax.ShapeDtypeStruct((_B, _N), input_ids.dtype),
            jax.ShapeDtypeStruct((_B, _N), attention_mask.dtype),
            jax.ShapeDtypeStruct((_B, _N), jnp.bool_),
        ],
    )(input_ids, attention_mask, rank8, thr8)
    return (out_ids, out_mask, xmask)


_NW = 32
_RW = _B // _NW
_CH = 32
_NCH = _RW // _CH


def _sc_body(ids_hbm, attn_hbm, p_hbm, oid_hbm, omask_hbm, xw_hbm,
             ids_v, attn_v, p_v, xw_v):
    wid = lax.axis_index("s") * 2 + lax.axis_index("c")
    base_row = wid * _RW
    iota16 = lax.iota(jnp.int32, 16)
    zeros16 = iota16 * 0
    ones16 = zeros16 + 1
    tok16 = zeros16 + _MASK_TOKEN

    def chunk(g, carry):
        row0 = base_row + g * _CH
        pltpu.sync_copy(ids_hbm.at[pl.ds(row0, _CH)], ids_v)
        pltpu.sync_copy(attn_hbm.at[pl.ds(row0, _CH)], attn_v)
        pltpu.sync_copy(p_hbm.at[pl.ds(row0, _CH)], p_v)
        for r in range(_CH):
            xw_v[r, pl.ds(0, 16)] = zeros16
            xw_v[r, pl.ds(16, 16)] = zeros16
        for grp in range(_CH // 16):
            cv = zeros16
            for m in range(16):
                r = grp * 16 + m
                acc = attn_v[r, pl.ds(0, 16)]
                for v in range(1, 8):
                    acc = acc + attn_v[r, pl.ds(16 * v, 16)]
                for sh in (8, 4, 2, 1):
                    rot = (iota16 + sh) & 15
                    acc = acc + lax.gather(
                        acc, rot[:, None],
                        lax.GatherDimensionNumbers(
                            offset_dims=(), collapsed_slice_dims=(0,),
                            start_index_map=(0,)),
                        slice_sizes=(1,),
                        mode=lax.GatherScatterMode.PROMISE_IN_BOUNDS)
                cv = jnp.where(iota16 == m, acc, cv)
            cm1 = jnp.minimum(cv, _N - 1)
            rows16 = iota16 + grp * 16
            pw = plsc.load_gather(p_v, [rows16, cm1])
            for m in range(_MASK_SIZE):
                pm = lax.shift_right_logical(pw, 8 * m) & 255
                plsc.store_scatter(ids_v, [rows16, pm], tok16)
                plsc.store_scatter(attn_v, [rows16, pm], zeros16)
                plsc.addupdate_scatter(
                    xw_v, [rows16, lax.shift_right_logical(pm, 2)],
                    lax.shift_left(ones16, 8 * (pm & 3)))
        pltpu.sync_copy(ids_v, oid_hbm.at[pl.ds(row0, _CH)])
        pltpu.sync_copy(attn_v, omask_hbm.at[pl.ds(row0, _CH)])
        pltpu.sync_copy(xw_v, xw_hbm.at[pl.ds(row0, _CH)])
        return carry

    lax.fori_loop(0, _NCH, chunk, 0)


@functools.lru_cache(maxsize=1)
def _sc_call():
    mesh = plsc.VectorSubcoreMesh(core_axis_name="c", subcore_axis_name="s")
    return pl.kernel(
        _sc_body,
        mesh=mesh,
        compiler_params=pltpu.CompilerParams(needs_layout_passes=False),
        out_type=[
            jax.ShapeDtypeStruct((_B, _N), jnp.int32),
            jax.ShapeDtypeStruct((_B, _N), jnp.int32),
            jax.ShapeDtypeStruct((_B, _N // 4), jnp.int32),
        ],
        scratch_types=[
            pltpu.VMEM((_CH, _N), jnp.int32),
            pltpu.VMEM((_CH, _N), jnp.int32),
            pltpu.VMEM((_CH, _N), jnp.int32),
            pltpu.VMEM((_CH, _N // 4), jnp.int32),
        ],
    )


@functools.lru_cache(maxsize=1)
def _p_table():
    u = _np_uniform_key42((_B, _N))
    order = np.argsort(-u, axis=1, kind="stable")
    rank = np.empty((_B, _N), np.int64)
    rank[np.arange(_B)[:, None], order] = np.arange(_N)[None, :]
    big = 10**6
    mr = np.full((_MASK_SIZE, _B), big, np.int64)
    mp = np.zeros((_MASK_SIZE, _B), np.int64)
    p = np.empty((_B, _N), np.int64)
    first4 = 0 | (1 << 8) | (2 << 16) | (3 << 24)
    for c in range(1, _N + 1):
        rn = rank[:, c - 1].copy()
        pn = np.full(_B, c - 1, np.int64)
        for k in range(_MASK_SIZE):
            swap = rn < mr[k]
            mr_k = np.where(swap, rn, mr[k])
            rn, mr[k] = np.where(swap, mr[k], rn), mr_k
            mp_k = np.where(swap, pn, mp[k])
            pn, mp[k] = np.where(swap, mp[k], pn), mp_k
        if c <= _MASK_SIZE:
            p[:, c - 1] = first4
        else:
            p[:, c - 1] = mp[0] | (mp[1] << 8) | (mp[2] << 16) | (mp[3] << 24)
    return jnp.asarray(p.astype(np.uint32).view(np.int32))


def _sc_kernel(input_ids, attention_mask):
    ptab = _p_table()
    oid, omask, xw = _sc_call()(input_ids, attention_mask, ptab)
    xmask = lax.bitcast_convert_type(xw, jnp.int8).reshape(_B, _N).astype(jnp.bool_)
    return (oid, omask, xmask)


def kernel(input_ids, attention_mask):
    return _sc_kernel(input_ids, attention_mask)


# speedup vs baseline: 1.3617x; 1.3617x over previous
import functools

import numpy as np
import jax
import jax.numpy as jnp
from jax import lax
from jax.experimental import pallas as pl
from jax.experimental.pallas import tpu as pltpu
from jax.experimental.pallas import tpu_sc as plsc

_MASK_SIZE = 4
_MASK_TOKEN = 14
_B, _N = 16384, 128
_BR = 1024


def _np_uniform_key42(shape):
    n = int(np.prod(shape))
    k0 = np.uint32(0)
    k1 = np.uint32(42)
    ks2 = np.uint32(k0 ^ k1 ^ np.uint32(0x1BD11BDA))
    x0 = np.zeros(n, dtype=np.uint32)
    x1 = np.arange(n, dtype=np.uint32)

    def rotl(x, r):
        return ((x << np.uint32(r)) | (x >> np.uint32(32 - r))).astype(np.uint32)

    def rounds(x0, x1, rots):
        for r in rots:
            x0 = (x0 + x1).astype(np.uint32)
            x1 = rotl(x1, r)
            x1 = x1 ^ x0
        return x0, x1

    ra, rb = (13, 15, 26, 6), (17, 29, 16, 24)
    x0 = (x0 + k0).astype(np.uint32)
    x1 = (x1 + k1).astype(np.uint32)
    x0, x1 = rounds(x0, x1, ra)
    x0 = (x0 + k1).astype(np.uint32); x1 = (x1 + ks2 + np.uint32(1)).astype(np.uint32)
    x0, x1 = rounds(x0, x1, rb)
    x0 = (x0 + ks2).astype(np.uint32); x1 = (x1 + k0 + np.uint32(2)).astype(np.uint32)
    x0, x1 = rounds(x0, x1, ra)
    x0 = (x0 + k0).astype(np.uint32); x1 = (x1 + k1 + np.uint32(3)).astype(np.uint32)
    x0, x1 = rounds(x0, x1, rb)
    x0 = (x0 + k1).astype(np.uint32); x1 = (x1 + ks2 + np.uint32(4)).astype(np.uint32)
    x0, x1 = rounds(x0, x1, ra)
    x0 = (x0 + ks2).astype(np.uint32); x1 = (x1 + k0 + np.uint32(5)).astype(np.uint32)
    bits = x0 ^ x1
    fbits = ((bits >> np.uint32(9)) | np.uint32(0x3F800000)).view(np.float32)
    return (fbits - np.float32(1.0)).reshape(shape)


@functools.lru_cache(maxsize=1)
def _tables():
    u = _np_uniform_key42((_B, _N))
    order = np.argsort(-u, axis=1, kind="stable")
    rank = np.empty((_B, _N), np.int32)
    rank[np.arange(_B)[:, None], order] = np.arange(_N)[None, :]
    big = 10**6
    m = np.full((_MASK_SIZE, _B), big, np.int64)
    thr = np.empty((_B, _N), np.int64)
    for c in range(1, _N + 1):
        x = rank[:, c - 1].astype(np.int64)
        for k in range(_MASK_SIZE):
            lo = np.minimum(m[k], x)
            x = np.maximum(m[k], x)
            m[k] = lo
        thr[:, c - 1] = np.where(m[_MASK_SIZE - 1] >= big, _N - 1, m[_MASK_SIZE - 1])
    return jnp.asarray(rank.astype(np.int8)), jnp.asarray(thr.astype(np.int8))


def _body(ids_ref, attn_ref, rank_ref, thr_ref, oid_ref, omask_ref, xm_ref):
    a = attn_ref[...]
    ones = jnp.ones((_N, _N), dtype=jnp.bfloat16)
    dn = (((1,), (0,)), ((), ()))
    csum = lax.dot_general(a.astype(jnp.bfloat16), ones, dn,
                           preferred_element_type=jnp.float32)
    c = jnp.minimum(csum.astype(jnp.int32) + 1, _N)
    pos = lax.broadcasted_iota(jnp.int32, (_BR, _N), 1)
    tsel = jnp.where(pos == c - 1, thr_ref[...].astype(jnp.bfloat16), jnp.bfloat16(0))
    thr_b = lax.dot_general(tsel, ones, dn,
                            preferred_element_type=jnp.float32).astype(jnp.int32)
    r = rank_ref[...].astype(jnp.int32)
    xm = ((pos < c) & (r <= thr_b)) | ((pos >= c) & (pos < _MASK_SIZE))
    oid_ref[...] = jnp.where(xm, _MASK_TOKEN, ids_ref[...])
    omask_ref[...] = jnp.where(xm, 0, a)
    xm_ref[...] = xm


def _tc_kernel(input_ids, attention_mask):
    rank8, thr8 = _tables()
    spec = pl.BlockSpec((_BR, _N), lambda i: (i, 0))
    out_ids, out_mask, xmask = pl.pallas_call(
        _body,
        grid=(_B // _BR,),
        in_specs=[spec, spec, spec, spec],
        out_specs=[spec, spec, spec],
        out_shape=[
            jax.ShapeDtypeStruct((_B, _N), input_ids.dtype),
            jax.ShapeDtypeStruct((_B, _N), attention_mask.dtype),
            jax.ShapeDtypeStruct((_B, _N), jnp.bool_),
        ],
    )(input_ids, attention_mask, rank8, thr8)
    return (out_ids, out_mask, xmask)


_NW = 32
_RW = _B // _NW
_CH = 64
_NCH = _RW // _CH


def _sc_body(ids_hbm, attn_hbm, p_hbm, oid_hbm, omask_hbm, xw_hbm,
             ids_v, attn_v, p_v, xw_v, isem0, isem1, osem0, osem1):
    wid = lax.axis_index("s") * 2 + lax.axis_index("c")
    base_row = wid * _RW
    iota16 = lax.iota(jnp.int32, 16)
    zeros16 = iota16 * 0
    ones16 = zeros16 + 1
    tok16 = zeros16 + _MASK_TOKEN
    isems = (isem0, isem1)
    osems = (osem0, osem1)

    def in_copies(g, slot):
        row0 = base_row + g * _CH
        s = isems[slot]
        return (
            pltpu.make_async_copy(ids_hbm.at[pl.ds(row0, _CH)], ids_v.at[slot], s),
            pltpu.make_async_copy(attn_hbm.at[pl.ds(row0, _CH)], attn_v.at[slot], s),
            pltpu.make_async_copy(p_hbm.at[pl.ds(row0, _CH)], p_v.at[slot], s),
        )

    def out_copies(g, slot):
        row0 = base_row + g * _CH
        s = osems[slot]
        return (
            pltpu.make_async_copy(ids_v.at[slot], oid_hbm.at[pl.ds(row0, _CH)], s),
            pltpu.make_async_copy(attn_v.at[slot], omask_hbm.at[pl.ds(row0, _CH)], s),
            pltpu.make_async_copy(xw_v.at[slot], xw_hbm.at[pl.ds(row0, _CH)], s),
        )

    def issue(copies):
        for cp in copies:
            cp.start()

    def wait(copies):
        for cp in copies:
            cp.wait()

    def compute(slot):
        sl16 = zeros16 + slot
        for r in range(_CH):
            xw_v[slot, r, pl.ds(0, 16)] = zeros16
            xw_v[slot, r, pl.ds(16, 16)] = zeros16
        for grp in range(_CH // 16):
            cv = zeros16
            for m in range(16):
                r = grp * 16 + m
                acc = attn_v[slot, r, pl.ds(0, 16)]
                for v in range(1, 8):
                    acc = acc + attn_v[slot, r, pl.ds(16 * v, 16)]
                for sh in (8, 4, 2, 1):
                    rot = (iota16 + sh) & 15
                    acc = acc + lax.gather(
                        acc, rot[:, None],
                        lax.GatherDimensionNumbers(
                            offset_dims=(), collapsed_slice_dims=(0,),
                            start_index_map=(0,)),
                        slice_sizes=(1,),
                        mode=lax.GatherScatterMode.PROMISE_IN_BOUNDS)
                cv = jnp.where(iota16 == m, acc, cv)
            cm1 = jnp.minimum(cv, _N - 1)
            rows16 = iota16 + grp * 16
            pw = plsc.load_gather(p_v, [sl16, rows16, cm1])
            for m in range(_MASK_SIZE):
                pm = lax.shift_right_logical(pw, 8 * m) & 255
                plsc.store_scatter(ids_v, [sl16, rows16, pm], tok16)
                plsc.store_scatter(attn_v, [sl16, rows16, pm], zeros16)
                plsc.addupdate_scatter(
                    xw_v, [sl16, rows16, lax.shift_right_logical(pm, 2)],
                    lax.shift_left(ones16, 8 * (pm & 3)))

    issue(in_copies(0, 0))

    def pair(i, carry):
        g0 = i * 2
        g1 = g0 + 1
        wait(in_copies(g0, 0))

        @pl.when(i > 0)
        def _():
            wait(out_copies(g0 - 1, 1))

        issue(in_copies(g1, 1))
        compute(0)
        issue(out_copies(g0, 0))
        wait(in_copies(g1, 1))
        compute(1)
        wait(out_copies(g0, 0))

        @pl.when(g0 + 2 < _NCH)
        def _():
            issue(in_copies(g0 + 2, 0))

        issue(out_copies(g1, 1))
        return carry

    lax.fori_loop(0, _NCH // 2, pair, 0)
    wait(out_copies(_NCH - 1, 1))


@functools.lru_cache(maxsize=1)
def _sc_call():
    mesh = plsc.VectorSubcoreMesh(core_axis_name="c", subcore_axis_name="s")
    return pl.kernel(
        _sc_body,
        mesh=mesh,
        compiler_params=pltpu.CompilerParams(needs_layout_passes=False),
        out_type=[
            jax.ShapeDtypeStruct((_B, _N), jnp.int32),
            jax.ShapeDtypeStruct((_B, _N), jnp.int32),
            jax.ShapeDtypeStruct((_B, _N // 4), jnp.int32),
        ],
        scratch_types=[
            pltpu.VMEM((2, _CH, _N), jnp.int32),
            pltpu.VMEM((2, _CH, _N), jnp.int32),
            pltpu.VMEM((2, _CH, _N), jnp.int32),
            pltpu.VMEM((2, _CH, _N // 4), jnp.int32),
            pltpu.SemaphoreType.DMA,
            pltpu.SemaphoreType.DMA,
            pltpu.SemaphoreType.DMA,
            pltpu.SemaphoreType.DMA,
        ],
    )


@functools.lru_cache(maxsize=1)
def _p_table():
    u = _np_uniform_key42((_B, _N))
    order = np.argsort(-u, axis=1, kind="stable")
    rank = np.empty((_B, _N), np.int64)
    rank[np.arange(_B)[:, None], order] = np.arange(_N)[None, :]
    big = 10**6
    mr = np.full((_MASK_SIZE, _B), big, np.int64)
    mp = np.zeros((_MASK_SIZE, _B), np.int64)
    p = np.empty((_B, _N), np.int64)
    first4 = 0 | (1 << 8) | (2 << 16) | (3 << 24)
    for c in range(1, _N + 1):
        rn = rank[:, c - 1].copy()
        pn = np.full(_B, c - 1, np.int64)
        for k in range(_MASK_SIZE):
            swap = rn < mr[k]
            mr_k = np.where(swap, rn, mr[k])
            rn, mr[k] = np.where(swap, mr[k], rn), mr_k
            mp_k = np.where(swap, pn, mp[k])
            pn, mp[k] = np.where(swap, mp[k], pn), mp_k
        if c <= _MASK_SIZE:
            p[:, c - 1] = first4
        else:
            p[:, c - 1] = mp[0] | (mp[1] << 8) | (mp[2] << 16) | (mp[3] << 24)
    return jnp.asarray(p.astype(np.uint32).view(np.int32))


def _sc_kernel(input_ids, attention_mask):
    ptab = _p_table()
    oid, omask, xw = _sc_call()(input_ids, attention_mask, ptab)
    xmask = lax.bitcast_convert_type(xw, jnp.int8).reshape(_B, _N).astype(jnp.bool_)
    return (oid, omask, xmask)


def kernel(input_ids, attention_mask):
    return _sc_kernel(input_ids, attention_mask)


# speedup vs baseline: 2.0136x; 1.4787x over previous
import functools

import numpy as np
import jax
import jax.numpy as jnp
from jax import lax
from jax.experimental import pallas as pl
from jax.experimental.pallas import tpu as pltpu
from jax.experimental.pallas import tpu_sc as plsc

_MASK_SIZE = 4
_MASK_TOKEN = 14
_B, _N = 16384, 128
_BR = 1024


def _np_uniform_key42(shape):
    n = int(np.prod(shape))
    k0 = np.uint32(0)
    k1 = np.uint32(42)
    ks2 = np.uint32(k0 ^ k1 ^ np.uint32(0x1BD11BDA))
    x0 = np.zeros(n, dtype=np.uint32)
    x1 = np.arange(n, dtype=np.uint32)

    def rotl(x, r):
        return ((x << np.uint32(r)) | (x >> np.uint32(32 - r))).astype(np.uint32)

    def rounds(x0, x1, rots):
        for r in rots:
            x0 = (x0 + x1).astype(np.uint32)
            x1 = rotl(x1, r)
            x1 = x1 ^ x0
        return x0, x1

    ra, rb = (13, 15, 26, 6), (17, 29, 16, 24)
    x0 = (x0 + k0).astype(np.uint32)
    x1 = (x1 + k1).astype(np.uint32)
    x0, x1 = rounds(x0, x1, ra)
    x0 = (x0 + k1).astype(np.uint32); x1 = (x1 + ks2 + np.uint32(1)).astype(np.uint32)
    x0, x1 = rounds(x0, x1, rb)
    x0 = (x0 + ks2).astype(np.uint32); x1 = (x1 + k0 + np.uint32(2)).astype(np.uint32)
    x0, x1 = rounds(x0, x1, ra)
    x0 = (x0 + k0).astype(np.uint32); x1 = (x1 + k1 + np.uint32(3)).astype(np.uint32)
    x0, x1 = rounds(x0, x1, rb)
    x0 = (x0 + k1).astype(np.uint32); x1 = (x1 + ks2 + np.uint32(4)).astype(np.uint32)
    x0, x1 = rounds(x0, x1, ra)
    x0 = (x0 + ks2).astype(np.uint32); x1 = (x1 + k0 + np.uint32(5)).astype(np.uint32)
    bits = x0 ^ x1
    fbits = ((bits >> np.uint32(9)) | np.uint32(0x3F800000)).view(np.float32)
    return (fbits - np.float32(1.0)).reshape(shape)


@functools.lru_cache(maxsize=1)
def _tables():
    u = _np_uniform_key42((_B, _N))
    order = np.argsort(-u, axis=1, kind="stable")
    rank = np.empty((_B, _N), np.int32)
    rank[np.arange(_B)[:, None], order] = np.arange(_N)[None, :]
    big = 10**6
    m = np.full((_MASK_SIZE, _B), big, np.int64)
    thr = np.empty((_B, _N), np.int64)
    for c in range(1, _N + 1):
        x = rank[:, c - 1].astype(np.int64)
        for k in range(_MASK_SIZE):
            lo = np.minimum(m[k], x)
            x = np.maximum(m[k], x)
            m[k] = lo
        thr[:, c - 1] = np.where(m[_MASK_SIZE - 1] >= big, _N - 1, m[_MASK_SIZE - 1])
    return jnp.asarray(rank.astype(np.int8)), jnp.asarray(thr.astype(np.int8))


def _body(ids_ref, attn_ref, rank_ref, thr_ref, oid_ref, omask_ref, xm_ref):
    a = attn_ref[...]
    ones = jnp.ones((_N, _N), dtype=jnp.bfloat16)
    dn = (((1,), (0,)), ((), ()))
    csum = lax.dot_general(a.astype(jnp.bfloat16), ones, dn,
                           preferred_element_type=jnp.float32)
    c = jnp.minimum(csum.astype(jnp.int32) + 1, _N)
    pos = lax.broadcasted_iota(jnp.int32, (_BR, _N), 1)
    tsel = jnp.where(pos == c - 1, thr_ref[...].astype(jnp.bfloat16), jnp.bfloat16(0))
    thr_b = lax.dot_general(tsel, ones, dn,
                            preferred_element_type=jnp.float32).astype(jnp.int32)
    r = rank_ref[...].astype(jnp.int32)
    xm = ((pos < c) & (r <= thr_b)) | ((pos >= c) & (pos < _MASK_SIZE))
    oid_ref[...] = jnp.where(xm, _MASK_TOKEN, ids_ref[...])
    omask_ref[...] = jnp.where(xm, 0, a)
    xm_ref[...] = xm


def _tc_kernel(input_ids, attention_mask):
    rank8, thr8 = _tables()
    spec = pl.BlockSpec((_BR, _N), lambda i: (i, 0))
    out_ids, out_mask, xmask = pl.pallas_call(
        _body,
        grid=(_B // _BR,),
        in_specs=[spec, spec, spec, spec],
        out_specs=[spec, spec, spec],
        out_shape=[
            jax.ShapeDtypeStruct((_B, _N), input_ids.dtype),
            jax.ShapeDtypeStruct((_B, _N), attention_mask.dtype),
            jax.ShapeDtypeStruct((_B, _N), jnp.bool_),
        ],
    )(input_ids, attention_mask, rank8, thr8)
    return (out_ids, out_mask, xmask)


_NW = 32
_RW = _B // _NW
_CH = 64
_NCH = _RW // _CH


def _sc_body(ids_hbm, attn_hbm, p_hbm, oid_hbm, omask_hbm, xw_hbm,
             ids_v, attn_v, p_v, xw_v, isem0, isem1, osem0, osem1):
    wid = lax.axis_index("s") * 2 + lax.axis_index("c")
    base_row = wid * _RW
    iota16 = lax.iota(jnp.int32, 16)
    zeros16 = iota16 * 0
    ones16 = zeros16 + 1
    tok16 = zeros16 + _MASK_TOKEN
    isems = (isem0, isem1)
    osems = (osem0, osem1)

    def in_copies(g, slot):
        row0 = base_row + g * _CH
        s = isems[slot]
        return (
            pltpu.make_async_copy(ids_hbm.at[pl.ds(row0, _CH)], ids_v.at[slot], s),
            pltpu.make_async_copy(attn_hbm.at[pl.ds(row0, _CH)], attn_v.at[slot], s),
            pltpu.make_async_copy(p_hbm.at[pl.ds(row0, _CH)], p_v.at[slot], s),
        )

    def out_copies(g, slot):
        row0 = base_row + g * _CH
        s = osems[slot]
        return (
            pltpu.make_async_copy(ids_v.at[slot], oid_hbm.at[pl.ds(row0, _CH)], s),
            pltpu.make_async_copy(attn_v.at[slot], omask_hbm.at[pl.ds(row0, _CH)], s),
            pltpu.make_async_copy(xw_v.at[slot], xw_hbm.at[pl.ds(row0, _CH)], s),
        )

    def issue(copies):
        for cp in copies:
            cp.start()

    def wait(copies):
        for cp in copies:
            cp.wait()

    def compute(slot):
        sl16 = zeros16 + slot
        for r in range(_CH):
            for v in range(8):
                xw_v[slot, r, pl.ds(16 * v, 16)] = zeros16
        for grp in range(_CH // 16):
            cv = zeros16
            for m in range(16):
                r = grp * 16 + m
                acc = attn_v[slot, r, pl.ds(0, 16)]
                for v in range(1, 8):
                    acc = acc + attn_v[slot, r, pl.ds(16 * v, 16)]
                for sh in (8, 4, 2, 1):
                    rot = (iota16 + sh) & 15
                    acc = acc + lax.gather(
                        acc, rot[:, None],
                        lax.GatherDimensionNumbers(
                            offset_dims=(), collapsed_slice_dims=(0,),
                            start_index_map=(0,)),
                        slice_sizes=(1,),
                        mode=lax.GatherScatterMode.PROMISE_IN_BOUNDS)
                cv = jnp.where(iota16 == m, acc, cv)
            cm1 = jnp.minimum(cv, _N - 1)
            rows16 = iota16 + grp * 16
            pw = plsc.load_gather(p_v, [sl16, rows16, cm1])
            for m in range(_MASK_SIZE):
                pm = lax.shift_right_logical(pw, 8 * m) & 255
                plsc.store_scatter(ids_v, [sl16, rows16, pm], tok16)
                plsc.store_scatter(attn_v, [sl16, rows16, pm], zeros16)
                plsc.store_scatter(xw_v, [sl16, rows16, pm], ones16)

    issue(in_copies(0, 0))

    def pair(i, carry):
        g0 = i * 2
        g1 = g0 + 1
        wait(in_copies(g0, 0))

        @pl.when(i > 0)
        def _():
            wait(out_copies(g0 - 1, 1))

        issue(in_copies(g1, 1))
        compute(0)
        issue(out_copies(g0, 0))
        wait(in_copies(g1, 1))
        compute(1)
        wait(out_copies(g0, 0))

        @pl.when(g0 + 2 < _NCH)
        def _():
            issue(in_copies(g0 + 2, 0))

        issue(out_copies(g1, 1))
        return carry

    lax.fori_loop(0, _NCH // 2, pair, 0)
    wait(out_copies(_NCH - 1, 1))


@functools.lru_cache(maxsize=1)
def _sc_call():
    mesh = plsc.VectorSubcoreMesh(core_axis_name="c", subcore_axis_name="s")
    return pl.kernel(
        _sc_body,
        mesh=mesh,
        compiler_params=pltpu.CompilerParams(needs_layout_passes=False),
        out_type=[
            jax.ShapeDtypeStruct((_B, _N), jnp.int32),
            jax.ShapeDtypeStruct((_B, _N), jnp.int32),
            jax.ShapeDtypeStruct((_B, _N), jnp.int32),
        ],
        scratch_types=[
            pltpu.VMEM((2, _CH, _N), jnp.int32),
            pltpu.VMEM((2, _CH, _N), jnp.int32),
            pltpu.VMEM((2, _CH, _N), jnp.int32),
            pltpu.VMEM((2, _CH, _N), jnp.int32),
            pltpu.SemaphoreType.DMA,
            pltpu.SemaphoreType.DMA,
            pltpu.SemaphoreType.DMA,
            pltpu.SemaphoreType.DMA,
        ],
    )


@functools.lru_cache(maxsize=1)
def _p_table():
    u = _np_uniform_key42((_B, _N))
    order = np.argsort(-u, axis=1, kind="stable")
    rank = np.empty((_B, _N), np.int64)
    rank[np.arange(_B)[:, None], order] = np.arange(_N)[None, :]
    big = 10**6
    mr = np.full((_MASK_SIZE, _B), big, np.int64)
    mp = np.zeros((_MASK_SIZE, _B), np.int64)
    p = np.empty((_B, _N), np.int64)
    first4 = 0 | (1 << 8) | (2 << 16) | (3 << 24)
    for c in range(1, _N + 1):
        rn = rank[:, c - 1].copy()
        pn = np.full(_B, c - 1, np.int64)
        for k in range(_MASK_SIZE):
            swap = rn < mr[k]
            mr_k = np.where(swap, rn, mr[k])
            rn, mr[k] = np.where(swap, mr[k], rn), mr_k
            mp_k = np.where(swap, pn, mp[k])
            pn, mp[k] = np.where(swap, mp[k], pn), mp_k
        if c <= _MASK_SIZE:
            p[:, c - 1] = first4
        else:
            p[:, c - 1] = mp[0] | (mp[1] << 8) | (mp[2] << 16) | (mp[3] << 24)
    return jnp.asarray(p.astype(np.uint32).view(np.int32))


def _sc_kernel(input_ids, attention_mask):
    ptab = _p_table()
    oid, omask, xw = _sc_call()(input_ids, attention_mask, ptab)
    return (oid, omask, xw.astype(jnp.bool_))


def kernel(input_ids, attention_mask):
    return _sc_kernel(input_ids, attention_mask)


# speedup vs baseline: 2.2409x; 1.1129x over previous
import functools

import numpy as np
import jax
import jax.numpy as jnp
from jax import lax
from jax.experimental import pallas as pl
from jax.experimental.pallas import tpu as pltpu
from jax.experimental.pallas import tpu_sc as plsc

_MASK_SIZE = 4
_MASK_TOKEN = 14
_B, _N = 16384, 128
_BR = 1024


def _np_uniform_key42(shape):
    n = int(np.prod(shape))
    k0 = np.uint32(0)
    k1 = np.uint32(42)
    ks2 = np.uint32(k0 ^ k1 ^ np.uint32(0x1BD11BDA))
    x0 = np.zeros(n, dtype=np.uint32)
    x1 = np.arange(n, dtype=np.uint32)

    def rotl(x, r):
        return ((x << np.uint32(r)) | (x >> np.uint32(32 - r))).astype(np.uint32)

    def rounds(x0, x1, rots):
        for r in rots:
            x0 = (x0 + x1).astype(np.uint32)
            x1 = rotl(x1, r)
            x1 = x1 ^ x0
        return x0, x1

    ra, rb = (13, 15, 26, 6), (17, 29, 16, 24)
    x0 = (x0 + k0).astype(np.uint32)
    x1 = (x1 + k1).astype(np.uint32)
    x0, x1 = rounds(x0, x1, ra)
    x0 = (x0 + k1).astype(np.uint32); x1 = (x1 + ks2 + np.uint32(1)).astype(np.uint32)
    x0, x1 = rounds(x0, x1, rb)
    x0 = (x0 + ks2).astype(np.uint32); x1 = (x1 + k0 + np.uint32(2)).astype(np.uint32)
    x0, x1 = rounds(x0, x1, ra)
    x0 = (x0 + k0).astype(np.uint32); x1 = (x1 + k1 + np.uint32(3)).astype(np.uint32)
    x0, x1 = rounds(x0, x1, rb)
    x0 = (x0 + k1).astype(np.uint32); x1 = (x1 + ks2 + np.uint32(4)).astype(np.uint32)
    x0, x1 = rounds(x0, x1, ra)
    x0 = (x0 + ks2).astype(np.uint32); x1 = (x1 + k0 + np.uint32(5)).astype(np.uint32)
    bits = x0 ^ x1
    fbits = ((bits >> np.uint32(9)) | np.uint32(0x3F800000)).view(np.float32)
    return (fbits - np.float32(1.0)).reshape(shape)


@functools.lru_cache(maxsize=1)
def _tables():
    u = _np_uniform_key42((_B, _N))
    order = np.argsort(-u, axis=1, kind="stable")
    rank = np.empty((_B, _N), np.int32)
    rank[np.arange(_B)[:, None], order] = np.arange(_N)[None, :]
    big = 10**6
    m = np.full((_MASK_SIZE, _B), big, np.int64)
    thr = np.empty((_B, _N), np.int64)
    for c in range(1, _N + 1):
        x = rank[:, c - 1].astype(np.int64)
        for k in range(_MASK_SIZE):
            lo = np.minimum(m[k], x)
            x = np.maximum(m[k], x)
            m[k] = lo
        thr[:, c - 1] = np.where(m[_MASK_SIZE - 1] >= big, _N - 1, m[_MASK_SIZE - 1])
    return jnp.asarray(rank.astype(np.int8)), jnp.asarray(thr.astype(np.int8))


def _body(ids_ref, attn_ref, rank_ref, thr_ref, oid_ref, omask_ref, xm_ref):
    a = attn_ref[...]
    ones = jnp.ones((_N, _N), dtype=jnp.bfloat16)
    dn = (((1,), (0,)), ((), ()))
    csum = lax.dot_general(a.astype(jnp.bfloat16), ones, dn,
                           preferred_element_type=jnp.float32)
    c = jnp.minimum(csum.astype(jnp.int32) + 1, _N)
    pos = lax.broadcasted_iota(jnp.int32, (_BR, _N), 1)
    tsel = jnp.where(pos == c - 1, thr_ref[...].astype(jnp.bfloat16), jnp.bfloat16(0))
    thr_b = lax.dot_general(tsel, ones, dn,
                            preferred_element_type=jnp.float32).astype(jnp.int32)
    r = rank_ref[...].astype(jnp.int32)
    xm = ((pos < c) & (r <= thr_b)) | ((pos >= c) & (pos < _MASK_SIZE))
    oid_ref[...] = jnp.where(xm, _MASK_TOKEN, ids_ref[...])
    omask_ref[...] = jnp.where(xm, 0, a)
    xm_ref[...] = xm


def _body_mask(attn_ref, rank_ref, thr_ref, omask_ref, xm_ref):
    a = attn_ref[...]
    ones = jnp.ones((_N, _N), dtype=jnp.bfloat16)
    dn = (((1,), (0,)), ((), ()))
    csum = lax.dot_general(a.astype(jnp.bfloat16), ones, dn,
                           preferred_element_type=jnp.float32)
    c = jnp.minimum(csum.astype(jnp.int32) + 1, _N)
    pos = lax.broadcasted_iota(jnp.int32, (_BR, _N), 1)
    tsel = jnp.where(pos == c - 1, thr_ref[...].astype(jnp.bfloat16), jnp.bfloat16(0))
    thr_b = lax.dot_general(tsel, ones, dn,
                            preferred_element_type=jnp.float32).astype(jnp.int32)
    r = rank_ref[...].astype(jnp.int32)
    xm = ((pos < c) & (r <= thr_b)) | ((pos >= c) & (pos < _MASK_SIZE))
    omask_ref[...] = jnp.where(xm, 0, a)
    xm_ref[...] = xm


def _tc_mask_kernel(attention_mask):
    rank8, thr8 = _tables()
    spec = pl.BlockSpec((_BR, _N), lambda i: (i, 0))
    out_mask, xmask = pl.pallas_call(
        _body_mask,
        grid=(_B // _BR,),
        in_specs=[spec, spec, spec],
        out_specs=[spec, spec],
        out_shape=[
            jax.ShapeDtypeStruct((_B, _N), attention_mask.dtype),
            jax.ShapeDtypeStruct((_B, _N), jnp.bool_),
        ],
    )(attention_mask, rank8, thr8)
    return (out_mask, xmask)


def _tc_kernel(input_ids, attention_mask):
    rank8, thr8 = _tables()
    spec = pl.BlockSpec((_BR, _N), lambda i: (i, 0))
    out_ids, out_mask, xmask = pl.pallas_call(
        _body,
        grid=(_B // _BR,),
        in_specs=[spec, spec, spec, spec],
        out_specs=[spec, spec, spec],
        out_shape=[
            jax.ShapeDtypeStruct((_B, _N), input_ids.dtype),
            jax.ShapeDtypeStruct((_B, _N), attention_mask.dtype),
            jax.ShapeDtypeStruct((_B, _N), jnp.bool_),
        ],
    )(input_ids, attention_mask, rank8, thr8)
    return (out_ids, out_mask, xmask)


_NW = 32
_RW = _B // _NW
_CH = 64
_NCH = _RW // _CH


def _sc_body(ids_hbm, attn_hbm, p_hbm, oid_hbm, omask_hbm, xw_hbm,
             ids_v, attn_v, p_v, xw_v, isem0, isem1, osem0, osem1):
    wid = lax.axis_index("s") * 2 + lax.axis_index("c")
    base_row = wid * _RW
    iota16 = lax.iota(jnp.int32, 16)
    zeros16 = iota16 * 0
    ones16 = zeros16 + 1
    tok16 = zeros16 + _MASK_TOKEN
    isems = (isem0, isem1)
    osems = (osem0, osem1)

    def in_copies(g, slot):
        row0 = base_row + g * _CH
        s = isems[slot]
        return (
            pltpu.make_async_copy(ids_hbm.at[pl.ds(row0, _CH)], ids_v.at[slot], s),
            pltpu.make_async_copy(attn_hbm.at[pl.ds(row0, _CH)], attn_v.at[slot], s),
            pltpu.make_async_copy(p_hbm.at[pl.ds(row0, _CH)], p_v.at[slot], s),
        )

    def out_copies(g, slot):
        row0 = base_row + g * _CH
        s = osems[slot]
        return (
            pltpu.make_async_copy(ids_v.at[slot], oid_hbm.at[pl.ds(row0, _CH)], s),
            pltpu.make_async_copy(attn_v.at[slot], omask_hbm.at[pl.ds(row0, _CH)], s),
            pltpu.make_async_copy(xw_v.at[slot], xw_hbm.at[pl.ds(row0, _CH)], s),
        )

    def issue(copies):
        for cp in copies:
            cp.start()

    def wait(copies):
        for cp in copies:
            cp.wait()

    def compute(slot):
        sl16 = zeros16 + slot
        for r in range(_CH):
            for v in range(8):
                xw_v[slot, r, pl.ds(16 * v, 16)] = zeros16
        for grp in range(_CH // 16):
            cv = zeros16
            for m in range(16):
                r = grp * 16 + m
                acc = attn_v[slot, r, pl.ds(0, 16)]
                for v in range(1, 8):
                    acc = acc + attn_v[slot, r, pl.ds(16 * v, 16)]
                for sh in (8, 4, 2, 1):
                    rot = (iota16 + sh) & 15
                    acc = acc + lax.gather(
                        acc, rot[:, None],
                        lax.GatherDimensionNumbers(
                            offset_dims=(), collapsed_slice_dims=(0,),
                            start_index_map=(0,)),
                        slice_sizes=(1,),
                        mode=lax.GatherScatterMode.PROMISE_IN_BOUNDS)
                cv = jnp.where(iota16 == m, acc, cv)
            cm1 = jnp.minimum(cv, _N - 1)
            rows16 = iota16 + grp * 16
            pw = plsc.load_gather(p_v, [sl16, rows16, cm1])
            for m in range(_MASK_SIZE):
                pm = lax.shift_right_logical(pw, 8 * m) & 255
                plsc.store_scatter(ids_v, [sl16, rows16, pm], tok16)
                plsc.store_scatter(attn_v, [sl16, rows16, pm], zeros16)
                plsc.store_scatter(xw_v, [sl16, rows16, pm], ones16)

    issue(in_copies(0, 0))

    def pair(i, carry):
        g0 = i * 2
        g1 = g0 + 1
        wait(in_copies(g0, 0))

        @pl.when(i > 0)
        def _():
            wait(out_copies(g0 - 1, 1))

        issue(in_copies(g1, 1))
        compute(0)
        issue(out_copies(g0, 0))
        wait(in_copies(g1, 1))
        compute(1)
        wait(out_copies(g0, 0))

        @pl.when(g0 + 2 < _NCH)
        def _():
            issue(in_copies(g0 + 2, 0))

        issue(out_copies(g1, 1))
        return carry

    lax.fori_loop(0, _NCH // 2, pair, 0)
    wait(out_copies(_NCH - 1, 1))


@functools.lru_cache(maxsize=1)
def _sc_call():
    mesh = plsc.VectorSubcoreMesh(core_axis_name="c", subcore_axis_name="s")
    return pl.kernel(
        _sc_body,
        mesh=mesh,
        compiler_params=pltpu.CompilerParams(needs_layout_passes=False),
        out_type=[
            jax.ShapeDtypeStruct((_B, _N), jnp.int32),
            jax.ShapeDtypeStruct((_B, _N), jnp.int32),
            jax.ShapeDtypeStruct((_B, _N), jnp.int32),
        ],
        scratch_types=[
            pltpu.VMEM((2, _CH, _N), jnp.int32),
            pltpu.VMEM((2, _CH, _N), jnp.int32),
            pltpu.VMEM((2, _CH, _N), jnp.int32),
            pltpu.VMEM((2, _CH, _N), jnp.int32),
            pltpu.SemaphoreType.DMA,
            pltpu.SemaphoreType.DMA,
            pltpu.SemaphoreType.DMA,
            pltpu.SemaphoreType.DMA,
        ],
    )


@functools.lru_cache(maxsize=1)
def _p_table():
    u = _np_uniform_key42((_B, _N))
    order = np.argsort(-u, axis=1, kind="stable")
    rank = np.empty((_B, _N), np.int64)
    rank[np.arange(_B)[:, None], order] = np.arange(_N)[None, :]
    big = 10**6
    mr = np.full((_MASK_SIZE, _B), big, np.int64)
    mp = np.zeros((_MASK_SIZE, _B), np.int64)
    p = np.empty((_B, _N), np.int64)
    first4 = 0 | (1 << 8) | (2 << 16) | (3 << 24)
    for c in range(1, _N + 1):
        rn = rank[:, c - 1].copy()
        pn = np.full(_B, c - 1, np.int64)
        for k in range(_MASK_SIZE):
            swap = rn < mr[k]
            mr_k = np.where(swap, rn, mr[k])
            rn, mr[k] = np.where(swap, mr[k], rn), mr_k
            mp_k = np.where(swap, pn, mp[k])
            pn, mp[k] = np.where(swap, mp[k], pn), mp_k
        if c <= _MASK_SIZE:
            p[:, c - 1] = first4
        else:
            p[:, c - 1] = mp[0] | (mp[1] << 8) | (mp[2] << 16) | (mp[3] << 24)
    return jnp.asarray(p.astype(np.uint32).view(np.int32))


def _sc_ids_body(ids_hbm, attn_hbm, p_hbm, oid_hbm,
                 ids_v, attn_v, p_v, isem0, isem1, osem0, osem1):
    wid = lax.axis_index("s") * 2 + lax.axis_index("c")
    base_row = wid * _RW
    iota16 = lax.iota(jnp.int32, 16)
    zeros16 = iota16 * 0
    tok16 = zeros16 + _MASK_TOKEN
    isems = (isem0, isem1)
    osems = (osem0, osem1)

    def in_copies(g, slot):
        row0 = base_row + g * _CH
        s = isems[slot]
        return (
            pltpu.make_async_copy(ids_hbm.at[pl.ds(row0, _CH)], ids_v.at[slot], s),
            pltpu.make_async_copy(attn_hbm.at[pl.ds(row0, _CH)], attn_v.at[slot], s),
            pltpu.make_async_copy(p_hbm.at[pl.ds(row0, _CH)], p_v.at[slot], s),
        )

    def out_copies(g, slot):
        row0 = base_row + g * _CH
        return (
            pltpu.make_async_copy(ids_v.at[slot], oid_hbm.at[pl.ds(row0, _CH)],
                                  osems[slot]),
        )

    def issue(copies):
        for cp in copies:
            cp.start()

    def wait(copies):
        for cp in copies:
            cp.wait()

    def compute(slot):
        sl16 = zeros16 + slot
        for grp in range(_CH // 16):
            cv = zeros16
            for m in range(16):
                r = grp * 16 + m
                acc = attn_v[slot, r, pl.ds(0, 16)]
                for v in range(1, 8):
                    acc = acc + attn_v[slot, r, pl.ds(16 * v, 16)]
                for sh in (8, 4, 2, 1):
                    rot = (iota16 + sh) & 15
                    acc = acc + lax.gather(
                        acc, rot[:, None],
                        lax.GatherDimensionNumbers(
                            offset_dims=(), collapsed_slice_dims=(0,),
                            start_index_map=(0,)),
                        slice_sizes=(1,),
                        mode=lax.GatherScatterMode.PROMISE_IN_BOUNDS)
                cv = jnp.where(iota16 == m, acc, cv)
            cm1 = jnp.minimum(cv, _N - 1)
            rows16 = iota16 + grp * 16
            pw = plsc.load_gather(p_v, [sl16, rows16, cm1])
            for m in range(_MASK_SIZE):
                pm = lax.shift_right_logical(pw, 8 * m) & 255
                plsc.store_scatter(ids_v, [sl16, rows16, pm], tok16)

    issue(in_copies(0, 0))

    def pair(i, carry):
        g0 = i * 2
        g1 = g0 + 1
        wait(in_copies(g0, 0))

        @pl.when(i > 0)
        def _():
            wait(out_copies(g0 - 1, 1))

        issue(in_copies(g1, 1))
        compute(0)
        issue(out_copies(g0, 0))
        wait(in_copies(g1, 1))
        compute(1)
        wait(out_copies(g0, 0))

        @pl.when(g0 + 2 < _NCH)
        def _():
            issue(in_copies(g0 + 2, 0))

        issue(out_copies(g1, 1))
        return carry

    lax.fori_loop(0, _NCH // 2, pair, 0)
    wait(out_copies(_NCH - 1, 1))


@functools.lru_cache(maxsize=1)
def _sc_ids_call():
    mesh = plsc.VectorSubcoreMesh(core_axis_name="c", subcore_axis_name="s")
    return pl.kernel(
        _sc_ids_body,
        mesh=mesh,
        compiler_params=pltpu.CompilerParams(needs_layout_passes=False),
        out_type=[
            jax.ShapeDtypeStruct((_B, _N), jnp.int32),
        ],
        scratch_types=[
            pltpu.VMEM((2, _CH, _N), jnp.int32),
            pltpu.VMEM((2, _CH, _N), jnp.int32),
            pltpu.VMEM((2, _CH, _N), jnp.int32),
            pltpu.SemaphoreType.DMA,
            pltpu.SemaphoreType.DMA,
            pltpu.SemaphoreType.DMA,
            pltpu.SemaphoreType.DMA,
        ],
    )


def _sc_kernel(input_ids, attention_mask):
    ptab = _p_table()
    oid, omask, xw = _sc_call()(input_ids, attention_mask, ptab)
    return (oid, omask, xw.astype(jnp.bool_))


def _hybrid_kernel(input_ids, attention_mask):
    ptab = _p_table()
    (oid,) = _sc_ids_call()(input_ids, attention_mask, ptab)
    out_mask, xmask = _tc_mask_kernel(attention_mask)
    return (oid, out_mask, xmask)


def kernel(input_ids, attention_mask):
    return _hybrid_kernel(input_ids, attention_mask)
